# baseline (speedup 1.0000x reference)
import functools
import math

import jax
import jax.numpy as jnp
from jax import lax
from jax.experimental import pallas as pl
from jax.experimental.pallas import tpu as pltpu
from jax.experimental.pallas import tpu_sc as plsc

R_MAX = 5.0
NUM_RADIAL = 8
HIDDEN = 16
P = 5
N_NODES = 10000
N_EDGES = 160000
MUL1 = HIDDEN // 2
SH0 = 0.28209479177387814
SH1 = 0.4886025119029199
INV_H = 1.0 / math.sqrt(HIDDEN)

NC = 2
NS = 16
NW = NC * NS
EPW = N_EDGES // NW
CK = 40
CH = EPW // CK
NPAD = 10240
NPW = NPAD // NS

BE = 2000
NB = N_EDGES // BE

_mesh = plsc.VectorSubcoreMesh(core_axis_name="c", subcore_axis_name="s")


NT0 = 624
LW = 128


def _gather_body(table_hbm, idx_hbm, out_hbm, idx_v, rows_v, table_sh, sem):
    c = lax.axis_index("c")
    s = lax.axis_index("s")
    wid = s * NC + c
    pltpu.sync_copy(table_hbm.at[pl.ds(s * NT0, NT0)],
                    table_sh.at[pl.ds(s * NT0, NT0)])

    @pl.when(s == NS - 1)
    def _tail():
        pltpu.sync_copy(table_hbm.at[pl.ds(NS * NT0, N_NODES - NS * NT0)],
                        table_sh.at[pl.ds(NS * NT0, N_NODES - NS * NT0)])

    pltpu.sync_copy(idx_hbm.at[wid], idx_v)
    plsc.subcore_barrier()

    def chunk(j, _):
        pltpu.async_copy(table_sh.at[idx_v.at[j]], rows_v, sem).wait()
        pltpu.sync_copy(rows_v, out_hbm.at[pl.ds(wid * EPW + j * CK, CK)])
        return 0

    lax.fori_loop(0, CH, chunk, 0)


_gather = pl.kernel(
    _gather_body,
    out_type=jax.ShapeDtypeStruct((N_EDGES, LW), jnp.float32),
    mesh=_mesh,
    scratch_types=[
        pltpu.VMEM((CH, CK), jnp.int32),
        pltpu.VMEM((CK, LW), jnp.float32),
        pltpu.VMEM_SHARED((N_NODES, LW), jnp.float32),
        pltpu.SemaphoreType.DMA,
    ],
)


FW = HIDDEN + 3 * MUL1


def _scatter_body(feats_hbm, idx_hbm, zeros_hbm, out_hbm, idx_v, buf_v, acc_sh, sem):
    c = lax.axis_index("c")
    s = lax.axis_index("s")
    wid = s * NC + c
    pltpu.sync_copy(zeros_hbm.at[pl.ds(s * NPW, NPW)], acc_sh.at[pl.ds(s * NPW, NPW)])
    pltpu.sync_copy(idx_hbm.at[wid], idx_v)
    plsc.subcore_barrier()

    def chunk(j, _):
        pltpu.sync_copy(feats_hbm.at[pl.ds(wid * EPW + j * CK, CK)], buf_v)
        pltpu.sync_copy(buf_v, acc_sh.at[idx_v.at[j]], add=True)
        return 0

    lax.fori_loop(0, CH, chunk, 0)
    plsc.subcore_barrier()
    pltpu.sync_copy(acc_sh.at[pl.ds(s * NPW, NPW)], out_hbm.at[c, pl.ds(s * NPW, NPW)])


_scatter = pl.kernel(
    _scatter_body,
    out_type=jax.ShapeDtypeStruct((NC, NPAD, LW), jnp.float32),
    mesh=_mesh,
    scratch_types=[
        pltpu.VMEM((CH, CK), jnp.int32),
        pltpu.VMEM((CK, LW), jnp.float32),
        pltpu.VMEM_SHARED((NPAD, LW), jnp.float32),
        pltpu.SemaphoreType.DMA,
    ],
)


def _edge_body(geom_ref, nf_ref, w1_ref, w2_ref, out_ref):
    g = geom_ref[...]
    d = g[:, 0:1]
    eps = jnp.finfo(jnp.float32).eps
    x = jnp.minimum(d * (1.0 / R_MAX), 1.0)
    x_p = x * x
    x_p = x_p * x_p * x
    cutoff = 1.0 - (P + 1) * x_p + P * x_p * x
    freq = (lax.broadcasted_iota(jnp.int32, (1, NUM_RADIAL), 1)
            .astype(jnp.float32) + 1.0) * math.pi
    dpos = jnp.maximum(d, 0.0)
    scaled = dpos * (freq * (1.0 / R_MAX))
    safe = jnp.maximum(dpos, eps)
    bessel = jnp.sin(scaled) / safe
    bessel = jnp.where(dpos == 0.0, freq * (1.0 / R_MAX), bessel)
    remb = (math.sqrt(2.0 / R_MAX)) * cutoff * bessel

    h = remb @ w1_ref[...] * (1.0 / math.sqrt(NUM_RADIAL))
    h = h * (1.0 / (1.0 + jnp.exp(-h)))
    w = h @ w2_ref[...] * (1.0 / 8.0)

    inv_d = 1.0 / jnp.maximum(d, eps)
    sh_y = SH1 * g[:, 2:3] * inv_d
    sh_z = SH1 * g[:, 3:4] * inv_d
    sh_x = SH1 * g[:, 1:2] * inv_d
    sh1 = jnp.concatenate([sh_y, sh_z, sh_x], axis=1)

    nf = nf_ref[:, :HIDDEN]
    out0 = jnp.zeros((g.shape[0], HIDDEN), jnp.float32)
    tmp1 = jnp.zeros((g.shape[0], MUL1), jnp.float32)
    for i in range(HIDDEN):
        col = nf[:, i:i + 1]
        out0 = out0 + col * w[:, i * HIDDEN:(i + 1) * HIDDEN]
        tmp1 = tmp1 + col * w[:, HIDDEN * HIDDEN + i * MUL1:
                              HIDDEN * HIDDEN + (i + 1) * MUL1]
    out0 = (INV_H * SH0) * out0
    tmp1 = INV_H * tmp1
    parts = [out0]
    for j in range(MUL1):
        parts.append(tmp1[:, j:j + 1] * sh1)
    parts.append(jnp.zeros((g.shape[0], LW - FW), jnp.float32))
    out_ref[...] = jnp.concatenate(parts, axis=1)


def _edge_tc(geom, nf, W1, W2):
    return pl.pallas_call(
        _edge_body,
        grid=(NB,),
        in_specs=[
            pl.BlockSpec((BE, 4), lambda i: (i, 0)),
            pl.BlockSpec((BE, LW), lambda i: (i, 0)),
            pl.BlockSpec((NUM_RADIAL, 64), lambda i: (0, 0)),
            pl.BlockSpec((64, HIDDEN * HIDDEN + HIDDEN * MUL1), lambda i: (0, 0)),
        ],
        out_specs=pl.BlockSpec((BE, LW), lambda i: (i, 0)),
        out_shape=jax.ShapeDtypeStruct((N_EDGES, LW), jnp.float32),
    )(geom, nf, W1, W2)


BN = 2000


def _node_body(p_ref, wexp_ref, o_ref):
    a = (p_ref[0] + p_ref[1])[:, :FW]
    mix = jnp.dot(a * a, wexp_ref[...],
                  preferred_element_type=jnp.float32)
    mix = mix * (1.0 / math.sqrt(HIDDEN + MUL1))
    o_ref[...] = a + jnp.concatenate(
        [mix, jnp.zeros((a.shape[0], 3 * MUL1), jnp.float32)], axis=1)


def _node_tc(partials, wexp):
    return pl.pallas_call(
        _node_body,
        grid=(N_NODES // BN,),
        in_specs=[
            pl.BlockSpec((NC, BN, LW), lambda i: (0, i, 0)),
            pl.BlockSpec((FW, HIDDEN), lambda i: (0, 0)),
        ],
        out_specs=pl.BlockSpec((BN, FW), lambda i: (i, 0)),
        out_shape=jax.ShapeDtypeStruct((N_NODES, FW), jnp.float32),
    )(partials, wexp)


def kernel(node_attrs, edge_index, edge_vec, edge_len, W1, W2, Wmix):
    sender = edge_index[0].reshape(NW, CH, CK)
    receiver = edge_index[1].reshape(NW, CH, CK)
    table_pad = jnp.pad(node_attrs, ((0, 0), (0, LW - HIDDEN)))
    nf = _gather(table_pad, sender)
    geom = jnp.concatenate([edge_len[:, None], edge_vec], axis=1)
    feats = _edge_tc(geom, nf, W1, W2)
    zeros = jnp.zeros((NPAD, LW), jnp.float32)
    partials = _scatter(feats, receiver, zeros)
    wexp = jnp.concatenate(
        [Wmix[:HIDDEN], jnp.repeat(Wmix[HIDDEN:], 3, axis=0) * (1.0 / math.sqrt(3.0))],
        axis=0)
    return _node_tc(partials, wexp)


# speedup vs baseline: 5.5658x; 5.5658x over previous
import functools
import math

import jax
import jax.numpy as jnp
from jax import lax
from jax.experimental import pallas as pl
from jax.experimental.pallas import tpu as pltpu
from jax.experimental.pallas import tpu_sc as plsc

R_MAX = 5.0
NUM_RADIAL = 8
HIDDEN = 16
P = 5
N_NODES = 10000
N_EDGES = 160000
MUL1 = HIDDEN // 2
SH0 = 0.28209479177387814
SH1 = 0.4886025119029199
INV_H = 1.0 / math.sqrt(HIDDEN)

NC = 2
NS = 16
NW = NC * NS
EPW = N_EDGES // NW
CK = 40
CH = EPW // CK
NPAD = 10240
NPW = NPAD // NS

BE = 3200
NB = N_EDGES // BE

_mesh = plsc.VectorSubcoreMesh(core_axis_name="c", subcore_axis_name="s")


NT0 = 624
LW = 128


def _gather_body(table_hbm, idx_hbm, out_hbm, idx_v, rows_v, table_sh, sem):
    c = lax.axis_index("c")
    s = lax.axis_index("s")
    wid = s * NC + c
    pltpu.sync_copy(table_hbm.at[pl.ds(s * NT0, NT0)],
                    table_sh.at[pl.ds(s * NT0, NT0)])

    @pl.when(s == NS - 1)
    def _tail():
        pltpu.sync_copy(table_hbm.at[pl.ds(NS * NT0, N_NODES - NS * NT0)],
                        table_sh.at[pl.ds(NS * NT0, N_NODES - NS * NT0)])

    pltpu.sync_copy(idx_hbm.at[wid], idx_v)
    plsc.subcore_barrier()

    def chunk(j, _):
        pltpu.async_copy(table_sh.at[idx_v.at[j]], rows_v, sem).wait()
        pltpu.sync_copy(rows_v, out_hbm.at[pl.ds(wid * EPW + j * CK, CK)])
        return 0

    lax.fori_loop(0, CH, chunk, 0)


_gather = pl.kernel(
    _gather_body,
    out_type=jax.ShapeDtypeStruct((N_EDGES, LW), jnp.float32),
    mesh=_mesh,
    scratch_types=[
        pltpu.VMEM((CH, CK), jnp.int32),
        pltpu.VMEM((CK, LW), jnp.float32),
        pltpu.VMEM_SHARED((N_NODES, LW), jnp.float32),
        pltpu.SemaphoreType.DMA,
    ],
)


FW = HIDDEN + 3 * MUL1


def _scatter_body(feats_hbm, idx_hbm, zeros_hbm, out_hbm, idx_v, buf_v, acc_sh, sem):
    c = lax.axis_index("c")
    s = lax.axis_index("s")
    wid = s * NC + c
    pltpu.sync_copy(zeros_hbm.at[pl.ds(s * NPW, NPW)], acc_sh.at[pl.ds(s * NPW, NPW)])
    pltpu.sync_copy(idx_hbm.at[wid], idx_v)
    plsc.subcore_barrier()

    def chunk(j, _):
        pltpu.sync_copy(feats_hbm.at[pl.ds(wid * EPW + j * CK, CK)], buf_v)
        pltpu.sync_copy(buf_v, acc_sh.at[idx_v.at[j]], add=True)
        return 0

    lax.fori_loop(0, CH, chunk, 0)
    plsc.subcore_barrier()
    pltpu.sync_copy(acc_sh.at[pl.ds(s * NPW, NPW)], out_hbm.at[c, pl.ds(s * NPW, NPW)])


_scatter = pl.kernel(
    _scatter_body,
    out_type=jax.ShapeDtypeStruct((NC, NPAD, LW), jnp.float32),
    mesh=_mesh,
    scratch_types=[
        pltpu.VMEM((CH, CK), jnp.int32),
        pltpu.VMEM((CK, LW), jnp.float32),
        pltpu.VMEM_SHARED((NPAD, LW), jnp.float32),
        pltpu.SemaphoreType.DMA,
    ],
)


def _edge_body(geom_ref, nf_ref, w1t_ref, w2t_ref, out_ref):
    g = geom_ref[...]
    d = g[0:1, :]
    eps = jnp.finfo(jnp.float32).eps
    x = jnp.minimum(d * (1.0 / R_MAX), 1.0)
    x_p = x * x
    x_p = x_p * x_p * x
    cutoff = 1.0 - (P + 1) * x_p + P * x_p * x
    freq = (lax.broadcasted_iota(jnp.int32, (NUM_RADIAL, 1), 0)
            .astype(jnp.float32) + 1.0) * math.pi
    dpos = jnp.maximum(d, 0.0)
    scaled = dpos * (freq * (1.0 / R_MAX))
    safe = jnp.maximum(dpos, eps)
    bessel = jnp.sin(scaled) / safe
    bessel = jnp.where(dpos == 0.0,
                       jnp.broadcast_to(freq * (1.0 / R_MAX), bessel.shape),
                       bessel)
    remb = (math.sqrt(2.0 / R_MAX)) * cutoff * bessel

    h = jnp.dot(w1t_ref[...], remb,
                preferred_element_type=jnp.float32) * (1.0 / math.sqrt(NUM_RADIAL))
    h = h * (1.0 / (1.0 + jnp.exp(-h)))
    w = jnp.dot(w2t_ref[...], h,
                preferred_element_type=jnp.float32) * (1.0 / 8.0)

    inv_d = 1.0 / jnp.maximum(d, eps)
    sh_y = SH1 * g[2:3, :] * inv_d
    sh_z = SH1 * g[3:4, :] * inv_d
    sh_x = SH1 * g[1:2, :] * inv_d

    nft = jnp.transpose(nf_ref[...][:, :HIDDEN])
    out0 = jnp.zeros((HIDDEN, g.shape[1]), jnp.float32)
    tmp1 = jnp.zeros((MUL1, g.shape[1]), jnp.float32)
    for i in range(HIDDEN):
        row = nft[i:i + 1, :]
        out0 = out0 + row * w[i * HIDDEN:(i + 1) * HIDDEN, :]
        tmp1 = tmp1 + row * w[HIDDEN * HIDDEN + i * MUL1:
                              HIDDEN * HIDDEN + (i + 1) * MUL1, :]
    out0 = (INV_H * SH0) * out0
    tmp1 = INV_H * tmp1
    parts = [out0]
    for j in range(MUL1):
        parts.append(tmp1[j:j + 1, :] * sh_y)
        parts.append(tmp1[j:j + 1, :] * sh_z)
        parts.append(tmp1[j:j + 1, :] * sh_x)
    outT = jnp.concatenate(parts, axis=0)
    rows = jnp.transpose(outT)
    out_ref[...] = jnp.concatenate(
        [rows, jnp.zeros((g.shape[1], LW - FW), jnp.float32)], axis=1)


def _edge_tc(geomT, nf, W1, W2):
    return pl.pallas_call(
        _edge_body,
        grid=(NB,),
        in_specs=[
            pl.BlockSpec((4, BE), lambda i: (0, i)),
            pl.BlockSpec((BE, LW), lambda i: (i, 0)),
            pl.BlockSpec((64, NUM_RADIAL), lambda i: (0, 0)),
            pl.BlockSpec((HIDDEN * HIDDEN + HIDDEN * MUL1, 64), lambda i: (0, 0)),
        ],
        out_specs=pl.BlockSpec((BE, LW), lambda i: (i, 0)),
        out_shape=jax.ShapeDtypeStruct((N_EDGES, LW), jnp.float32),
    )(geomT, nf, W1.T, W2.T)


BN = 2000


def _node_body(p_ref, wexp_ref, o_ref):
    a = (p_ref[0] + p_ref[1])[:, :FW]
    mix = jnp.dot(a * a, wexp_ref[...],
                  preferred_element_type=jnp.float32)
    mix = mix * (1.0 / math.sqrt(HIDDEN + MUL1))
    o_ref[...] = a + jnp.concatenate(
        [mix, jnp.zeros((a.shape[0], 3 * MUL1), jnp.float32)], axis=1)


def _node_tc(partials, wexp):
    return pl.pallas_call(
        _node_body,
        grid=(N_NODES // BN,),
        in_specs=[
            pl.BlockSpec((NC, BN, LW), lambda i: (0, i, 0)),
            pl.BlockSpec((FW, HIDDEN), lambda i: (0, 0)),
        ],
        out_specs=pl.BlockSpec((BN, FW), lambda i: (i, 0)),
        out_shape=jax.ShapeDtypeStruct((N_NODES, FW), jnp.float32),
    )(partials, wexp)


def kernel(node_attrs, edge_index, edge_vec, edge_len, W1, W2, Wmix):
    sender = edge_index[0].reshape(NW, CH, CK)
    receiver = edge_index[1].reshape(NW, CH, CK)
    table_pad = jnp.pad(node_attrs, ((0, 0), (0, LW - HIDDEN)))
    nf = _gather(table_pad, sender)
    geomT = jnp.concatenate([edge_len[None, :], edge_vec.T], axis=0)
    feats = _edge_tc(geomT, nf, W1, W2)
    zeros = jnp.zeros((NPAD, LW), jnp.float32)
    partials = _scatter(feats, receiver, zeros)
    wexp = jnp.concatenate(
        [Wmix[:HIDDEN], jnp.repeat(Wmix[HIDDEN:], 3, axis=0) * (1.0 / math.sqrt(3.0))],
        axis=0)
    return _node_tc(partials, wexp)


# speedup vs baseline: 5.5751x; 1.0017x over previous
import functools
import math

import jax
import jax.numpy as jnp
from jax import lax
from jax.experimental import pallas as pl
from jax.experimental.pallas import tpu as pltpu
from jax.experimental.pallas import tpu_sc as plsc

R_MAX = 5.0
NUM_RADIAL = 8
HIDDEN = 16
P = 5
N_NODES = 10000
N_EDGES = 160000
MUL1 = HIDDEN // 2
SH0 = 0.28209479177387814
SH1 = 0.4886025119029199
INV_H = 1.0 / math.sqrt(HIDDEN)

NC = 2
NS = 16
NW = NC * NS
EPW = N_EDGES // NW
CK = 40
CH = EPW // CK
NPAD = 10240
NPW = NPAD // NS

BE = 3200
NB = N_EDGES // BE

_mesh = plsc.VectorSubcoreMesh(core_axis_name="c", subcore_axis_name="s")


NT0 = 624
LWG = 128
LWS = 128


def _gather_body(table_hbm, idx_hbm, out_hbm, idx_v, rows_v, table_sh, sem):
    c = lax.axis_index("c")
    s = lax.axis_index("s")
    wid = s * NC + c
    pltpu.sync_copy(table_hbm.at[pl.ds(s * NT0, NT0)],
                    table_sh.at[pl.ds(s * NT0, NT0)])

    @pl.when(s == NS - 1)
    def _tail():
        pltpu.sync_copy(table_hbm.at[pl.ds(NS * NT0, N_NODES - NS * NT0)],
                        table_sh.at[pl.ds(NS * NT0, N_NODES - NS * NT0)])

    pltpu.sync_copy(idx_hbm.at[wid], idx_v)
    plsc.subcore_barrier()

    def chunk(j, _):
        pltpu.async_copy(table_sh.at[idx_v.at[j]], rows_v, sem).wait()
        pltpu.sync_copy(rows_v, out_hbm.at[pl.ds(wid * EPW + j * CK, CK)])
        return 0

    lax.fori_loop(0, CH, chunk, 0)


_gather = pl.kernel(
    _gather_body,
    out_type=jax.ShapeDtypeStruct((N_EDGES, LWG), jnp.float32),
    mesh=_mesh,
    scratch_types=[
        pltpu.VMEM((CH, CK), jnp.int32),
        pltpu.VMEM((CK, LWG), jnp.float32),
        pltpu.VMEM_SHARED((N_NODES, LWG), jnp.float32),
        pltpu.SemaphoreType.DMA,
    ],
)


FW = HIDDEN + 3 * MUL1


def _scatter_body(feats_hbm, idx_hbm, zeros_hbm, out_hbm, idx_v, buf_v, acc_sh, sem):
    c = lax.axis_index("c")
    s = lax.axis_index("s")
    wid = s * NC + c
    pltpu.sync_copy(zeros_hbm.at[pl.ds(s * NPW, NPW)], acc_sh.at[pl.ds(s * NPW, NPW)])
    pltpu.sync_copy(idx_hbm.at[wid], idx_v)
    plsc.subcore_barrier()

    def chunk(j, _):
        pltpu.sync_copy(feats_hbm.at[pl.ds(wid * EPW + j * CK, CK)], buf_v)
        pltpu.sync_copy(buf_v, acc_sh.at[idx_v.at[j]], add=True)
        return 0

    lax.fori_loop(0, CH, chunk, 0)
    plsc.subcore_barrier()
    pltpu.sync_copy(acc_sh.at[pl.ds(s * NPW, NPW)], out_hbm.at[c, pl.ds(s * NPW, NPW)])


_scatter = pl.kernel(
    _scatter_body,
    out_type=jax.ShapeDtypeStruct((NC, NPAD, LWS), jnp.float32),
    mesh=_mesh,
    scratch_types=[
        pltpu.VMEM((CH, CK), jnp.int32),
        pltpu.VMEM((CK, LWS), jnp.float32),
        pltpu.VMEM_SHARED((NPAD, LWS), jnp.float32),
        pltpu.SemaphoreType.DMA,
    ],
)


def _edge_body(geom_ref, nf_ref, w1t_ref, w2t_ref, out_ref):
    g = geom_ref[...]
    d = g[0:1, :]
    eps = jnp.finfo(jnp.float32).eps
    x = jnp.minimum(d * (1.0 / R_MAX), 1.0)
    x_p = x * x
    x_p = x_p * x_p * x
    cutoff = 1.0 - (P + 1) * x_p + P * x_p * x
    freq = (lax.broadcasted_iota(jnp.int32, (NUM_RADIAL, 1), 0)
            .astype(jnp.float32) + 1.0) * math.pi
    dpos = jnp.maximum(d, 0.0)
    scaled = dpos * (freq * (1.0 / R_MAX))
    safe = jnp.maximum(dpos, eps)
    bessel = jnp.sin(scaled) / safe
    bessel = jnp.where(dpos == 0.0,
                       jnp.broadcast_to(freq * (1.0 / R_MAX), bessel.shape),
                       bessel)
    remb = (math.sqrt(2.0 / R_MAX)) * cutoff * bessel

    h = jnp.dot(w1t_ref[...], remb,
                preferred_element_type=jnp.float32) * (1.0 / math.sqrt(NUM_RADIAL))
    h = h * (1.0 / (1.0 + jnp.exp(-h)))
    w = jnp.dot(w2t_ref[...], h,
                preferred_element_type=jnp.float32) * (1.0 / 8.0)

    inv_d = 1.0 / jnp.maximum(d, eps)
    sh_y = SH1 * g[2:3, :] * inv_d
    sh_z = SH1 * g[3:4, :] * inv_d
    sh_x = SH1 * g[1:2, :] * inv_d

    nft = jnp.transpose(nf_ref[...][:, :HIDDEN])
    out0 = jnp.zeros((HIDDEN, g.shape[1]), jnp.float32)
    tmp1 = jnp.zeros((MUL1, g.shape[1]), jnp.float32)
    for i in range(HIDDEN):
        row = nft[i:i + 1, :]
        out0 = out0 + row * w[i * HIDDEN:(i + 1) * HIDDEN, :]
        tmp1 = tmp1 + row * w[HIDDEN * HIDDEN + i * MUL1:
                              HIDDEN * HIDDEN + (i + 1) * MUL1, :]
    out0 = (INV_H * SH0) * out0
    tmp1 = INV_H * tmp1
    parts = [out0]
    for j in range(MUL1):
        parts.append(tmp1[j:j + 1, :] * sh_y)
        parts.append(tmp1[j:j + 1, :] * sh_z)
        parts.append(tmp1[j:j + 1, :] * sh_x)
    outT = jnp.concatenate(parts, axis=0)
    rows = jnp.transpose(outT)
    out_ref[...] = jnp.concatenate(
        [rows, jnp.zeros((g.shape[1], LWS - FW), jnp.float32)], axis=1)


def _edge_tc(geomT, nf, W1, W2):
    return pl.pallas_call(
        _edge_body,
        grid=(NB,),
        in_specs=[
            pl.BlockSpec((4, BE), lambda i: (0, i)),
            pl.BlockSpec((BE, LWG), lambda i: (i, 0)),
            pl.BlockSpec((64, NUM_RADIAL), lambda i: (0, 0)),
            pl.BlockSpec((HIDDEN * HIDDEN + HIDDEN * MUL1, 64), lambda i: (0, 0)),
        ],
        out_specs=pl.BlockSpec((BE, LWS), lambda i: (i, 0)),
        out_shape=jax.ShapeDtypeStruct((N_EDGES, LWS), jnp.float32),
    )(geomT, nf, W1.T, W2.T)


BN = 2000


def _node_body(p_ref, wexp_ref, o_ref):
    a = (p_ref[0] + p_ref[1])[:, :FW]
    mix = jnp.dot(a * a, wexp_ref[...],
                  preferred_element_type=jnp.float32)
    mix = mix * (1.0 / math.sqrt(HIDDEN + MUL1))
    o_ref[...] = a + jnp.concatenate(
        [mix, jnp.zeros((a.shape[0], 3 * MUL1), jnp.float32)], axis=1)


def _node_tc(partials, wexp):
    return pl.pallas_call(
        _node_body,
        grid=(N_NODES // BN,),
        in_specs=[
            pl.BlockSpec((NC, BN, LWS), lambda i: (0, i, 0)),
            pl.BlockSpec((FW, HIDDEN), lambda i: (0, 0)),
        ],
        out_specs=pl.BlockSpec((BN, FW), lambda i: (i, 0)),
        out_shape=jax.ShapeDtypeStruct((N_NODES, FW), jnp.float32),
    )(partials, wexp)


def kernel(node_attrs, edge_index, edge_vec, edge_len, W1, W2, Wmix):
    sender = edge_index[0].reshape(NW, CH, CK)
    receiver = edge_index[1].reshape(NW, CH, CK)
    table_pad = jnp.pad(node_attrs, ((0, 0), (0, LWG - HIDDEN)))
    nf = _gather(table_pad, sender)
    geomT = jnp.concatenate([edge_len[None, :], edge_vec.T], axis=0)
    feats = _edge_tc(geomT, nf, W1, W2)
    zeros = jnp.zeros((NPAD, LWS), jnp.float32)
    partials = _scatter(feats, receiver, zeros)
    wexp = jnp.concatenate(
        [Wmix[:HIDDEN], jnp.repeat(Wmix[HIDDEN:], 3, axis=0) * (1.0 / math.sqrt(3.0))],
        axis=0)
    return _node_tc(partials, wexp)


# speedup vs baseline: 5.7448x; 1.0304x over previous
import functools
import math

import jax
import jax.numpy as jnp
from jax import lax
from jax.experimental import pallas as pl
from jax.experimental.pallas import tpu as pltpu
from jax.experimental.pallas import tpu_sc as plsc

R_MAX = 5.0
NUM_RADIAL = 8
HIDDEN = 16
P = 5
N_NODES = 10000
N_EDGES = 160000
MUL1 = HIDDEN // 2
SH0 = 0.28209479177387814
SH1 = 0.4886025119029199
INV_H = 1.0 / math.sqrt(HIDDEN)

NC = 2
NS = 16
NW = NC * NS
EPW = N_EDGES // NW
CK = 40
CH = EPW // CK
NPAD = 10240
NPW = NPAD // NS

BE = 6400
NB = N_EDGES // BE

_mesh = plsc.VectorSubcoreMesh(core_axis_name="c", subcore_axis_name="s")


NT0 = 624
LWG = 128
LWS = 128


def _gather_body(table_hbm, idx_hbm, out_hbm, idx_v, rows_v, table_sh, sem):
    c = lax.axis_index("c")
    s = lax.axis_index("s")
    wid = s * NC + c
    pltpu.sync_copy(table_hbm.at[pl.ds(s * NT0, NT0)],
                    table_sh.at[pl.ds(s * NT0, NT0)])

    @pl.when(s == NS - 1)
    def _tail():
        pltpu.sync_copy(table_hbm.at[pl.ds(NS * NT0, N_NODES - NS * NT0)],
                        table_sh.at[pl.ds(NS * NT0, N_NODES - NS * NT0)])

    pltpu.sync_copy(idx_hbm.at[wid], idx_v)
    plsc.subcore_barrier()

    def chunk(j, _):
        pltpu.async_copy(table_sh.at[idx_v.at[j]], rows_v, sem).wait()
        pltpu.sync_copy(rows_v, out_hbm.at[pl.ds(wid * EPW + j * CK, CK)])
        return 0

    lax.fori_loop(0, CH, chunk, 0)


_gather = pl.kernel(
    _gather_body,
    out_type=jax.ShapeDtypeStruct((N_EDGES, LWG), jnp.float32),
    mesh=_mesh,
    scratch_types=[
        pltpu.VMEM((CH, CK), jnp.int32),
        pltpu.VMEM((CK, LWG), jnp.float32),
        pltpu.VMEM_SHARED((N_NODES, LWG), jnp.float32),
        pltpu.SemaphoreType.DMA,
    ],
)


FW = HIDDEN + 3 * MUL1


def _scatter_body(feats_hbm, idx_hbm, zeros_hbm, out_hbm, idx_v, buf_v, acc_sh, sem):
    c = lax.axis_index("c")
    s = lax.axis_index("s")
    wid = s * NC + c
    pltpu.sync_copy(zeros_hbm.at[pl.ds(s * NPW, NPW)], acc_sh.at[pl.ds(s * NPW, NPW)])
    pltpu.sync_copy(idx_hbm.at[wid], idx_v)
    plsc.subcore_barrier()

    def chunk(j, _):
        pltpu.sync_copy(feats_hbm.at[pl.ds(wid * EPW + j * CK, CK)], buf_v)
        pltpu.sync_copy(buf_v, acc_sh.at[idx_v.at[j]], add=True)
        return 0

    lax.fori_loop(0, CH, chunk, 0)
    plsc.subcore_barrier()
    pltpu.sync_copy(acc_sh.at[pl.ds(s * NPW, NPW)], out_hbm.at[c, pl.ds(s * NPW, NPW)])


_scatter = pl.kernel(
    _scatter_body,
    out_type=jax.ShapeDtypeStruct((NC, NPAD, LWS), jnp.float32),
    mesh=_mesh,
    scratch_types=[
        pltpu.VMEM((CH, CK), jnp.int32),
        pltpu.VMEM((CK, LWS), jnp.float32),
        pltpu.VMEM_SHARED((NPAD, LWS), jnp.float32),
        pltpu.SemaphoreType.DMA,
    ],
)


def _edge_body(geom_ref, nf_ref, w1t_ref, w2t_ref, out_ref):
    g = geom_ref[...]
    d = g[0:1, :]
    eps = jnp.finfo(jnp.float32).eps
    x = jnp.minimum(d * (1.0 / R_MAX), 1.0)
    x_p = x * x
    x_p = x_p * x_p * x
    cutoff = 1.0 - (P + 1) * x_p + P * x_p * x
    freq = (lax.broadcasted_iota(jnp.int32, (NUM_RADIAL, 1), 0)
            .astype(jnp.float32) + 1.0) * math.pi
    dpos = jnp.maximum(d, 0.0)
    scaled = dpos * (freq * (1.0 / R_MAX))
    safe = jnp.maximum(dpos, eps)
    bessel = jnp.sin(scaled) / safe
    bessel = jnp.where(dpos == 0.0,
                       jnp.broadcast_to(freq * (1.0 / R_MAX), bessel.shape),
                       bessel)
    remb = (math.sqrt(2.0 / R_MAX)) * cutoff * bessel

    h = jnp.dot(w1t_ref[...], remb,
                preferred_element_type=jnp.float32) * (1.0 / math.sqrt(NUM_RADIAL))
    h = h * (1.0 / (1.0 + jnp.exp(-h)))
    w = jnp.dot(w2t_ref[...], h,
                preferred_element_type=jnp.float32) * (1.0 / 8.0)

    inv_d = 1.0 / jnp.maximum(d, eps)
    sh_y = SH1 * g[2:3, :] * inv_d
    sh_z = SH1 * g[3:4, :] * inv_d
    sh_x = SH1 * g[1:2, :] * inv_d

    nft = jnp.transpose(nf_ref[...][:, :HIDDEN])
    out0 = jnp.zeros((HIDDEN, g.shape[1]), jnp.float32)
    tmp1 = jnp.zeros((MUL1, g.shape[1]), jnp.float32)
    for i in range(HIDDEN):
        row = nft[i:i + 1, :]
        out0 = out0 + row * w[i * HIDDEN:(i + 1) * HIDDEN, :]
        tmp1 = tmp1 + row * w[HIDDEN * HIDDEN + i * MUL1:
                              HIDDEN * HIDDEN + (i + 1) * MUL1, :]
    out0 = (INV_H * SH0) * out0
    tmp1 = INV_H * tmp1
    parts = [out0]
    for j in range(MUL1):
        parts.append(tmp1[j:j + 1, :] * sh_y)
        parts.append(tmp1[j:j + 1, :] * sh_z)
        parts.append(tmp1[j:j + 1, :] * sh_x)
    outT = jnp.concatenate(parts, axis=0)
    rows = jnp.transpose(outT)
    out_ref[...] = jnp.concatenate(
        [rows, jnp.zeros((g.shape[1], LWS - FW), jnp.float32)], axis=1)


def _edge_tc(geomT, nf, W1, W2):
    return pl.pallas_call(
        _edge_body,
        grid=(NB,),
        in_specs=[
            pl.BlockSpec((4, BE), lambda i: (0, i)),
            pl.BlockSpec((BE, LWG), lambda i: (i, 0)),
            pl.BlockSpec((64, NUM_RADIAL), lambda i: (0, 0)),
            pl.BlockSpec((HIDDEN * HIDDEN + HIDDEN * MUL1, 64), lambda i: (0, 0)),
        ],
        out_specs=pl.BlockSpec((BE, LWS), lambda i: (i, 0)),
        out_shape=jax.ShapeDtypeStruct((N_EDGES, LWS), jnp.float32),
    )(geomT, nf, W1.T, W2.T)


BN = 2000


def _node_body(p_ref, wexp_ref, o_ref):
    a = (p_ref[0] + p_ref[1])[:, :FW]
    mix = jnp.dot(a * a, wexp_ref[...],
                  preferred_element_type=jnp.float32)
    mix = mix * (1.0 / math.sqrt(HIDDEN + MUL1))
    o_ref[...] = a + jnp.concatenate(
        [mix, jnp.zeros((a.shape[0], 3 * MUL1), jnp.float32)], axis=1)


def _node_tc(partials, wexp):
    return pl.pallas_call(
        _node_body,
        grid=(N_NODES // BN,),
        in_specs=[
            pl.BlockSpec((NC, BN, LWS), lambda i: (0, i, 0)),
            pl.BlockSpec((FW, HIDDEN), lambda i: (0, 0)),
        ],
        out_specs=pl.BlockSpec((BN, FW), lambda i: (i, 0)),
        out_shape=jax.ShapeDtypeStruct((N_NODES, FW), jnp.float32),
    )(partials, wexp)


def kernel(node_attrs, edge_index, edge_vec, edge_len, W1, W2, Wmix):
    sender = edge_index[0].reshape(NW, CH, CK)
    receiver = edge_index[1].reshape(NW, CH, CK)
    table_pad = jnp.pad(node_attrs, ((0, 0), (0, LWG - HIDDEN)))
    nf = _gather(table_pad, sender)
    geomT = jnp.concatenate([edge_len[None, :], edge_vec.T], axis=0)
    feats = _edge_tc(geomT, nf, W1, W2)
    zeros = jnp.zeros((NPAD, LWS), jnp.float32)
    partials = _scatter(feats, receiver, zeros)
    wexp = jnp.concatenate(
        [Wmix[:HIDDEN], jnp.repeat(Wmix[HIDDEN:], 3, axis=0) * (1.0 / math.sqrt(3.0))],
        axis=0)
    return _node_tc(partials, wexp)
